# baseline (speedup 1.0000x reference)
import functools
import math

import jax
import jax.numpy as jnp
import numpy as np
from jax import lax
from jax.experimental import pallas as pl
from jax.experimental.pallas import tpu as pltpu
from jax.experimental.pallas import tpu_sc as plsc

D_MODEL = 768
SEQ_LEN = 8192
BATCH = 4
NUM_CORES = 2
NUM_SUBCORES = 16
NUM_WORKERS = NUM_CORES * NUM_SUBCORES
POS_PER_WORKER = SEQ_LEN // NUM_WORKERS
CHUNK = 32
CHUNKS_PER_WORKER = POS_PER_WORKER // CHUNK
STEPS = CHUNKS_PER_WORKER * BATCH
LANES = 16
CBLK = 16
NBLK = D_MODEL // (LANES * CBLK)
SCALE = math.sqrt(D_MODEL)


def _pos_encoding(seq_len, d_model):
    pos = np.arange(seq_len, dtype=np.float32).reshape(-1, 1)
    dim = np.arange(0, d_model, 2, dtype=np.float32).reshape(1, -1)
    div = pos / 10000 ** (dim / d_model)
    pe = np.zeros((seq_len, d_model), np.float32)
    pe[:, 0::2] = np.sin(div)
    pe[:, 1::2] = np.cos(div)
    return pe


_PE = _pos_encoding(SEQ_LEN, D_MODEL)


def kernel(x, emb_table):
    pe = jnp.asarray(_PE.reshape(-1))
    xf = x.reshape(-1)
    mesh = plsc.VectorSubcoreMesh(core_axis_name="c", subcore_axis_name="s")

    @functools.partial(
        pl.kernel,
        mesh=mesh,
        out_type=jax.ShapeDtypeStruct((BATCH * SEQ_LEN, D_MODEL), jnp.float32),
        scratch_types=[
            pltpu.VMEM((BATCH, POS_PER_WORKER), jnp.int32),
            pltpu.VMEM((2, CHUNK * D_MODEL), jnp.float32),
            pltpu.VMEM((2, CHUNK, D_MODEL), jnp.float32),
            pltpu.SemaphoreType.DMA,
            pltpu.SemaphoreType.DMA,
            pltpu.SemaphoreType.DMA,
            pltpu.SemaphoreType.DMA,
            pltpu.SemaphoreType.DMA,
        ],
    )
    def emb_kernel(x_hbm, table_hbm, pe_hbm, out_hbm, idx_v, pe_v, rows_v,
                   sem_p, sem_g0, sem_g1, sem_w0, sem_w1):
        wid = lax.axis_index("s") * NUM_CORES + lax.axis_index("c")
        pos0 = wid * POS_PER_WORKER
        sem_g = (sem_g0, sem_g1)
        sem_w = (sem_w0, sem_w1)

        def gather_start(k, b, buf, sem):
            idx_slice = idx_v.at[b, pl.ds(k * CHUNK, CHUNK)]
            pltpu.async_copy(table_hbm.at[idx_slice], rows_v.at[buf], sem)

        def gather_wait(buf, sem):
            pltpu.make_async_copy(
                table_hbm.at[idx_v.at[0, pl.ds(0, CHUNK)]],
                rows_v.at[buf], sem).wait()

        def pe_start(k, buf):
            pltpu.async_copy(
                pe_hbm.at[pl.ds((pos0 + k * CHUNK) * D_MODEL, CHUNK * D_MODEL)],
                pe_v.at[buf], sem_p)

        def pe_wait(buf):
            pltpu.make_async_copy(pe_hbm.at[pl.ds(0, CHUNK * D_MODEL)],
                                  pe_v.at[buf], sem_p).wait()

        def write_start(k, b, buf, sem):
            pltpu.async_copy(
                rows_v.at[buf],
                out_hbm.at[pl.ds(b * SEQ_LEN + pos0 + k * CHUNK, CHUNK)],
                sem)

        def write_wait(buf, sem):
            pltpu.make_async_copy(rows_v.at[buf],
                                  out_hbm.at[pl.ds(0, CHUNK)],
                                  sem).wait()

        for b in range(BATCH):
            pltpu.sync_copy(
                x_hbm.at[pl.ds(b * SEQ_LEN + pos0, POS_PER_WORKER)],
                idx_v.at[b])

        pe_start(0, 0)
        gather_start(0, 0, 0, sem_g0)

        def step_body(s, carry):
            k = s // BATCH
            b = s % BATCH
            sn = s + 1
            kn = sn // BATCH
            bn = sn % BATCH
            par = s % 2
            parn = sn % 2
            kpar = k % 2

            @pl.when(s >= 1)
            def _():
                @pl.when(parn == 0)
                def _():
                    write_wait(0, sem_w0)

                @pl.when(parn == 1)
                def _():
                    write_wait(1, sem_w1)

            @pl.when(s < STEPS - 1)
            def _():
                @pl.when(parn == 0)
                def _():
                    gather_start(kn, bn, 0, sem_g0)

                @pl.when(parn == 1)
                def _():
                    gather_start(kn, bn, 1, sem_g1)

            @pl.when(b == 0)
            def _():
                @pl.when(kpar == 0)
                def _():
                    pe_wait(0)

                @pl.when(kpar == 1)
                def _():
                    pe_wait(1)

                @pl.when(k < CHUNKS_PER_WORKER - 1)
                def _():
                    @pl.when(kpar == 0)
                    def _():
                        pe_start(k + 1, 1)

                    @pl.when(kpar == 1)
                    def _():
                        pe_start(k + 1, 0)

            @pl.when(par == 0)
            def _():
                gather_wait(0, sem_g0)

            @pl.when(par == 1)
            def _():
                gather_wait(1, sem_g1)

            @plsc.parallel_loop(0, CHUNK, unroll=2)
            def _row(r):
                pbase = r * D_MODEL
                for c in range(D_MODEL // LANES):
                    off = pl.ds(c * LANES, LANES)
                    rows_v[par, r, off] = (
                        rows_v[par, r, off] * SCALE
                        + pe_v[kpar, pl.ds(pbase + c * LANES, LANES)])

            @pl.when(par == 0)
            def _():
                write_start(k, b, 0, sem_w0)

            @pl.when(par == 1)
            def _():
                write_start(k, b, 1, sem_w1)

            return carry

        lax.fori_loop(0, STEPS, step_body, 0)

        write_wait(1, sem_w1)

    out = emb_kernel(xf, emb_table, pe)
    return out.reshape(BATCH, SEQ_LEN, D_MODEL)


# speedup vs baseline: 1.1337x; 1.1337x over previous
import functools
import math

import jax
import jax.numpy as jnp
import numpy as np
from jax import lax
from jax.experimental import pallas as pl
from jax.experimental.pallas import tpu as pltpu
from jax.experimental.pallas import tpu_sc as plsc

D_MODEL = 768
SEQ_LEN = 8192
BATCH = 4
NUM_CORES = 2
NUM_SUBCORES = 16
NUM_WORKERS = NUM_CORES * NUM_SUBCORES
POS_PER_WORKER = SEQ_LEN // NUM_WORKERS
CHUNK = 32
CHUNKS_PER_WORKER = POS_PER_WORKER // CHUNK
STEPS = CHUNKS_PER_WORKER * BATCH
LANES = 16
NBUF = 3
SCALE = math.sqrt(D_MODEL)


def _pos_encoding(seq_len, d_model):
    pos = np.arange(seq_len, dtype=np.float32).reshape(-1, 1)
    dim = np.arange(0, d_model, 2, dtype=np.float32).reshape(1, -1)
    div = pos / 10000 ** (dim / d_model)
    pe = np.zeros((seq_len, d_model), np.float32)
    pe[:, 0::2] = np.sin(div)
    pe[:, 1::2] = np.cos(div)
    return pe


_PE = _pos_encoding(SEQ_LEN, D_MODEL)


def kernel(x, emb_table):
    pe = jnp.asarray(_PE.reshape(-1))
    xf = x.reshape(-1)
    mesh = plsc.VectorSubcoreMesh(core_axis_name="c", subcore_axis_name="s")

    @functools.partial(
        pl.kernel,
        mesh=mesh,
        out_type=jax.ShapeDtypeStruct((BATCH * SEQ_LEN, D_MODEL), jnp.float32),
        scratch_types=[
            pltpu.VMEM((BATCH, POS_PER_WORKER), jnp.int32),
            pltpu.VMEM((2, CHUNK * D_MODEL), jnp.float32),
            pltpu.VMEM((NBUF, CHUNK, D_MODEL), jnp.float32),
            pltpu.SemaphoreType.DMA,
            pltpu.SemaphoreType.DMA,
            pltpu.SemaphoreType.DMA,
            pltpu.SemaphoreType.DMA,
            pltpu.SemaphoreType.DMA,
            pltpu.SemaphoreType.DMA,
            pltpu.SemaphoreType.DMA,
        ],
    )
    def emb_kernel(x_hbm, table_hbm, pe_hbm, out_hbm, idx_v, pe_v, rows_v,
                   sem_p, sem_g0, sem_g1, sem_g2, sem_w0, sem_w1, sem_w2):
        wid = lax.axis_index("s") * NUM_CORES + lax.axis_index("c")
        pos0 = wid * POS_PER_WORKER
        sem_g = (sem_g0, sem_g1, sem_g2)
        sem_w = (sem_w0, sem_w1, sem_w2)

        def gather_start(k, b, buf):
            idx_slice = idx_v.at[b, pl.ds(k * CHUNK, CHUNK)]
            pltpu.async_copy(table_hbm.at[idx_slice], rows_v.at[buf],
                             sem_g[buf])

        def gather_wait(buf):
            pltpu.make_async_copy(
                table_hbm.at[idx_v.at[0, pl.ds(0, CHUNK)]],
                rows_v.at[buf], sem_g[buf]).wait()

        def pe_start(k, buf):
            pltpu.async_copy(
                pe_hbm.at[pl.ds((pos0 + k * CHUNK) * D_MODEL,
                                CHUNK * D_MODEL)],
                pe_v.at[buf], sem_p)

        def pe_wait(buf):
            pltpu.make_async_copy(pe_hbm.at[pl.ds(0, CHUNK * D_MODEL)],
                                  pe_v.at[buf], sem_p).wait()

        def write_start(k, b, buf):
            pltpu.async_copy(
                rows_v.at[buf],
                out_hbm.at[pl.ds(b * SEQ_LEN + pos0 + k * CHUNK, CHUNK)],
                sem_w[buf])

        def write_wait(buf):
            pltpu.make_async_copy(rows_v.at[buf],
                                  out_hbm.at[pl.ds(0, CHUNK)],
                                  sem_w[buf]).wait()

        def on_par(par, fn):
            for p in range(NBUF):
                @pl.when(par == p)
                def _(p=p):
                    fn(p)

        for b in range(BATCH):
            pltpu.sync_copy(
                x_hbm.at[pl.ds(b * SEQ_LEN + pos0, POS_PER_WORKER)],
                idx_v.at[b])

        pe_start(0, 0)
        gather_start(0, 0, 0)
        gather_start(0, 1, 1)

        def step_body(s, carry):
            k = s // BATCH
            b = s % BATCH
            par = s % NBUF
            kpar = k % 2

            on_par(par, lambda p: gather_wait(p))

            @pl.when(b == 0)
            def _():
                @pl.when(kpar == 0)
                def _():
                    pe_wait(0)

                @pl.when(kpar == 1)
                def _():
                    pe_wait(1)

                @pl.when(k < CHUNKS_PER_WORKER - 1)
                def _():
                    @pl.when(kpar == 0)
                    def _():
                        pe_start(k + 1, 1)

                    @pl.when(kpar == 1)
                    def _():
                        pe_start(k + 1, 0)

            @plsc.parallel_loop(0, CHUNK)
            def _row(r):
                pbase = r * D_MODEL
                for c in range(D_MODEL // LANES):
                    off = pl.ds(c * LANES, LANES)
                    rows_v[par, r, off] = (
                        rows_v[par, r, off] * SCALE
                        + pe_v[kpar, pl.ds(pbase + c * LANES, LANES)])

            @pl.when(s >= 1)
            def _():
                on_par((s + NBUF - 1) % NBUF, lambda p: write_wait(p))

            on_par(par, lambda p: write_start(k, b, p))

            @pl.when(s < STEPS - 2)
            def _():
                s2 = s + 2
                on_par(s2 % NBUF,
                       lambda p: gather_start(s2 // BATCH, s2 % BATCH, p))

            return carry

        lax.fori_loop(0, STEPS, step_body, 0)

        write_wait((STEPS - 1) % NBUF)

    out = emb_kernel(xf, emb_table, pe)
    return out.reshape(BATCH, SEQ_LEN, D_MODEL)


# speedup vs baseline: 1.1426x; 1.0078x over previous
import functools
import math

import jax
import jax.numpy as jnp
import numpy as np
from jax import lax
from jax.experimental import pallas as pl
from jax.experimental.pallas import tpu as pltpu
from jax.experimental.pallas import tpu_sc as plsc

D_MODEL = 768
SEQ_LEN = 8192
BATCH = 4
NUM_CORES = 2
NUM_SUBCORES = 16
NUM_WORKERS = NUM_CORES * NUM_SUBCORES
POS_PER_WORKER = SEQ_LEN // NUM_WORKERS
CHUNK = 16
CHUNKS_PER_WORKER = POS_PER_WORKER // CHUNK
STEPS = CHUNKS_PER_WORKER * BATCH
LANES = 16
NBUF = 4
SCALE = math.sqrt(D_MODEL)


def _pos_encoding(seq_len, d_model):
    pos = np.arange(seq_len, dtype=np.float32).reshape(-1, 1)
    dim = np.arange(0, d_model, 2, dtype=np.float32).reshape(1, -1)
    div = pos / 10000 ** (dim / d_model)
    pe = np.zeros((seq_len, d_model), np.float32)
    pe[:, 0::2] = np.sin(div)
    pe[:, 1::2] = np.cos(div)
    return pe


_PE = _pos_encoding(SEQ_LEN, D_MODEL)


def kernel(x, emb_table):
    pe = jnp.asarray(_PE.reshape(-1))
    xf = x.reshape(-1)
    mesh = plsc.VectorSubcoreMesh(core_axis_name="c", subcore_axis_name="s")

    @functools.partial(
        pl.kernel,
        mesh=mesh,
        out_type=jax.ShapeDtypeStruct((BATCH * SEQ_LEN, D_MODEL), jnp.float32),
        scratch_types=[
            pltpu.VMEM((BATCH, POS_PER_WORKER), jnp.int32),
            pltpu.VMEM((2, CHUNK * D_MODEL), jnp.float32),
            pltpu.VMEM((NBUF, CHUNK, D_MODEL), jnp.float32),
        ] + [pltpu.SemaphoreType.DMA] * (1 + 2 * NBUF),
    )
    def emb_kernel(x_hbm, table_hbm, pe_hbm, out_hbm, idx_v, pe_v, rows_v,
                   sem_p, *sems):
        wid = lax.axis_index("s") * NUM_CORES + lax.axis_index("c")
        pos0 = wid * POS_PER_WORKER
        sem_g = sems[:NBUF]
        sem_w = sems[NBUF:]

        def gather_start(k, b, buf):
            idx_slice = idx_v.at[b, pl.ds(k * CHUNK, CHUNK)]
            pltpu.async_copy(table_hbm.at[idx_slice], rows_v.at[buf],
                             sem_g[buf])

        def gather_wait(buf):
            pltpu.make_async_copy(
                table_hbm.at[idx_v.at[0, pl.ds(0, CHUNK)]],
                rows_v.at[buf], sem_g[buf]).wait()

        def pe_start(k, buf):
            pltpu.async_copy(
                pe_hbm.at[pl.ds((pos0 + k * CHUNK) * D_MODEL,
                                CHUNK * D_MODEL)],
                pe_v.at[buf], sem_p)

        def pe_wait(buf):
            pltpu.make_async_copy(pe_hbm.at[pl.ds(0, CHUNK * D_MODEL)],
                                  pe_v.at[buf], sem_p).wait()

        def write_start(k, b, buf):
            pltpu.async_copy(
                rows_v.at[buf],
                out_hbm.at[pl.ds(b * SEQ_LEN + pos0 + k * CHUNK, CHUNK)],
                sem_w[buf])

        def write_wait(buf):
            pltpu.make_async_copy(rows_v.at[buf],
                                  out_hbm.at[pl.ds(0, CHUNK)],
                                  sem_w[buf]).wait()

        def on_par(par, fn):
            for p in range(NBUF):
                @pl.when(par == p)
                def _(p=p):
                    fn(p)

        for b in range(BATCH):
            pltpu.sync_copy(
                x_hbm.at[pl.ds(b * SEQ_LEN + pos0, POS_PER_WORKER)],
                idx_v.at[b])

        pe_start(0, 0)
        for i in range(NBUF - 1):
            gather_start(i // BATCH, i % BATCH, i % NBUF)

        def step_body(s, carry):
            k = s // BATCH
            b = s % BATCH
            par = s % NBUF
            kpar = k % 2

            on_par(par, lambda p: gather_wait(p))

            @pl.when(b == 0)
            def _():
                @pl.when(kpar == 0)
                def _():
                    pe_wait(0)

                @pl.when(kpar == 1)
                def _():
                    pe_wait(1)

                @pl.when(k < CHUNKS_PER_WORKER - 1)
                def _():
                    @pl.when(kpar == 0)
                    def _():
                        pe_start(k + 1, 1)

                    @pl.when(kpar == 1)
                    def _():
                        pe_start(k + 1, 0)

            @plsc.parallel_loop(0, CHUNK)
            def _row(r):
                pbase = r * D_MODEL
                for c in range(D_MODEL // LANES):
                    off = pl.ds(c * LANES, LANES)
                    rows_v[par, r, off] = (
                        rows_v[par, r, off] * SCALE
                        + pe_v[kpar, pl.ds(pbase + c * LANES, LANES)])

            @pl.when(s >= 1)
            def _():
                on_par((s + NBUF - 1) % NBUF, lambda p: write_wait(p))

            on_par(par, lambda p: write_start(k, b, p))

            @pl.when(s < STEPS - (NBUF - 1))
            def _():
                s2 = s + NBUF - 1
                on_par(s2 % NBUF,
                       lambda p: gather_start(s2 // BATCH, s2 % BATCH, p))

            return carry

        lax.fori_loop(0, STEPS, step_body, 0)

        write_wait((STEPS - 1) % NBUF)

    out = emb_kernel(xf, emb_table, pe)
    return out.reshape(BATCH, SEQ_LEN, D_MODEL)
